# baseline (speedup 1.0000x reference)
import functools

import jax
import jax.numpy as jnp
from jax import lax
from jax.experimental import pallas as pl
from jax.experimental.pallas import tpu as pltpu
from jax.experimental.pallas import tpu_sc as plsc

B, C, R, K = 4, 192, 512, 15
NCH = 8
NSEL = 8
LANES = 16
CCH = 8
TASKS = B * (C // CCH)
NBUF = 8
RCHUNKS = R // LANES
UNROLL = 16


def _make_sc_call():
    info = plsc.get_sparse_core_info()
    nc, ns = info.num_cores, info.num_subcores
    nw = nc * ns
    assert TASKS % nw == 0
    tpw = TASKS // nw

    mesh = plsc.VectorSubcoreMesh(core_axis_name="c", subcore_axis_name="s")

    @functools.partial(
        pl.kernel,
        mesh=mesh,
        compiler_params=pltpu.CompilerParams(
            needs_layout_passes=False, use_tc_tiling_on_sc=True),
        out_type=jax.ShapeDtypeStruct((B, NCH * C, NSEL, R), jnp.float32),
        scratch_types=[
            pltpu.VMEM((NCH * LANES,), jnp.float32),
            pltpu.VMEM((LANES,), jnp.int32),
            pltpu.VMEM((LANES,), jnp.float32),
            pltpu.VMEM((NCH * LANES,), jnp.int32),
            pltpu.VMEM((NCH * LANES,), jnp.float32),
            pltpu.VMEM((K, CCH, R), jnp.float32),
            pltpu.VMEM((NBUF, NSEL, R), jnp.float32),
            pltpu.SemaphoreType.DMA,
            pltpu.SemaphoreType.DMA,
        ],
    )
    def sc_call(x_hbm, w_hbm, out_hbm, w_v, idxtab, wtab, coltab, wseltab,
                xin, ob, si, so):
        wid = lax.axis_index("s") * nc + lax.axis_index("c")

        task0 = wid * tpw
        b0 = task0 // (C // CCH)
        c00 = (task0 - b0 * (C // CCH)) * CCH
        for k in range(K):
            pltpu.async_copy(x_hbm.at[b0, k, pl.ds(c00, CCH)], xin.at[k], si)

        pltpu.sync_copy(w_hbm, w_v)

        iota = lax.iota(jnp.int32, LANES)
        n_vec = lax.bitwise_and(iota, NSEL - 1)

        for i in range(NCH):
            row = w_v[pl.ds(i * LANES, LANES)]
            rank = jnp.zeros((LANES,), jnp.int32)
            for j in range(K):
                wj = jnp.full((LANES,), row[j])
                cond = (wj < row) | ((wj == row) & (j < iota))
                rank = rank + cond.astype(jnp.int32)
            sel = rank < NSEL
            sig = 1.0 / (1.0 + jnp.exp(-row))
            plsc.store_scatter(idxtab, [rank], iota, mask=sel)
            plsc.store_scatter(wtab, [rank], sig, mask=sel)
            coltab[pl.ds(i * LANES, LANES)] = plsc.load_gather(idxtab, [n_vec])
            wseltab[pl.ds(i * LANES, LANES)] = plsc.load_gather(wtab, [n_vec])

        def task_body(t, carry):
            task = wid * tpw + t
            b = task // (C // CCH)
            c0 = (task - b * (C // CCH)) * CCH

            @pl.when(t > 0)
            def _():
                for k in range(K):
                    pltpu.async_copy(x_hbm.at[b, k, pl.ds(c0, CCH)],
                                     xin.at[k], si)

            for k in range(K):
                pltpu.make_async_copy(x_hbm.at[b, k, pl.ds(c0, CCH)],
                                      xin.at[k], si).wait()

            def slab_body(s, carry2):
                i = s // CCH
                c = s - i * CCH
                gs = t * (NCH * CCH) + s
                slot = lax.rem(gs, NBUF)
                ch = i * C + c0 + c

                @pl.when(gs >= NBUF)
                def _():
                    pltpu.make_async_copy(ob.at[slot], out_hbm.at[b, ch],
                                          so).wait()

                for n in range(NSEL):
                    sel_ix = jnp.full((LANES,), i * LANES + n, jnp.int32)
                    kv = plsc.load_gather(coltab, [sel_ix])
                    wv = plsc.load_gather(wseltab, [sel_ix])
                    k = kv[0]

                    @plsc.parallel_loop(0, RCHUNKS, unroll=UNROLL)
                    def rchunk(j, n=n, k=k, c=c, wv=wv, slot=slot):
                        v = xin[k, c, pl.ds(j * LANES, LANES)]
                        ob[slot, n, pl.ds(j * LANES, LANES)] = v * wv

                pltpu.async_copy(ob.at[slot], out_hbm.at[b, ch], so)
                return carry2

            lax.fori_loop(0, NCH * CCH, slab_body, 0)
            return carry

        lax.fori_loop(0, tpw, task_body, 0)

        for _ in range(NBUF):
            pltpu.make_async_copy(ob.at[0], out_hbm.at[0, 0], so).wait()

    return sc_call


_sc_call = _make_sc_call()


def kernel(X, partition_weights):
    wpad = jnp.concatenate(
        [partition_weights,
         jnp.full((NCH, LANES - K), jnp.inf, jnp.float32)], axis=1)
    xt = X.transpose(0, 3, 1, 2)
    out = _sc_call(xt, wpad.reshape(NCH * LANES))
    return out.transpose(0, 1, 3, 2)


# speedup vs baseline: 1.1830x; 1.1830x over previous
import functools

import jax
import jax.numpy as jnp
from jax import lax
from jax.experimental import pallas as pl
from jax.experimental.pallas import tpu as pltpu
from jax.experimental.pallas import tpu_sc as plsc

B, C, R, K = 4, 192, 512, 15
NCH = 8
NSEL = 8
LANES = 16
CCH = 8
TASKS = B * (C // CCH)
NBUF = 8
RCHUNKS = R // LANES
UNROLL = 8


def _make_sc_call():
    info = plsc.get_sparse_core_info()
    nc, ns = info.num_cores, info.num_subcores
    nw = nc * ns
    assert TASKS % nw == 0
    tpw = TASKS // nw

    mesh = plsc.VectorSubcoreMesh(core_axis_name="c", subcore_axis_name="s")

    @functools.partial(
        pl.kernel,
        mesh=mesh,
        compiler_params=pltpu.CompilerParams(
            needs_layout_passes=False, use_tc_tiling_on_sc=True),
        out_type=jax.ShapeDtypeStruct((B, NCH * C, NSEL, R), jnp.float32),
        scratch_types=[
            pltpu.VMEM((NCH * LANES,), jnp.float32),
            pltpu.VMEM((LANES,), jnp.int32),
            pltpu.VMEM((LANES,), jnp.float32),
            pltpu.VMEM((NCH * LANES,), jnp.int32),
            pltpu.VMEM((NCH * LANES,), jnp.float32),
            pltpu.VMEM((K, CCH, R), jnp.float32),
            pltpu.VMEM((NBUF, NSEL, R), jnp.float32),
            pltpu.SemaphoreType.DMA,
            pltpu.SemaphoreType.DMA,
        ],
    )
    def sc_call(x_hbm, w_hbm, out_hbm, w_v, idxtab, wtab, coltab, wseltab,
                xin, ob, si, so):
        wid = lax.axis_index("s") * nc + lax.axis_index("c")

        task0 = wid * tpw
        b0 = task0 // (C // CCH)
        c00 = (task0 - b0 * (C // CCH)) * CCH
        for k in range(K):
            pltpu.async_copy(x_hbm.at[b0, k, pl.ds(c00, CCH)], xin.at[k], si)

        pltpu.sync_copy(w_hbm, w_v)

        iota = lax.iota(jnp.int32, LANES)
        n_vec = lax.bitwise_and(iota, NSEL - 1)

        for i in range(NCH):
            row = w_v[pl.ds(i * LANES, LANES)]
            rank = jnp.zeros((LANES,), jnp.int32)
            for j in range(K):
                wj = jnp.full((LANES,), row[j])
                cond = (wj < row) | ((wj == row) & (j < iota))
                rank = rank + cond.astype(jnp.int32)
            sel = rank < NSEL
            sig = 1.0 / (1.0 + jnp.exp(-row))
            plsc.store_scatter(idxtab, [rank], iota, mask=sel)
            plsc.store_scatter(wtab, [rank], sig, mask=sel)
            coltab[pl.ds(i * LANES, LANES)] = plsc.load_gather(idxtab, [n_vec])
            wseltab[pl.ds(i * LANES, LANES)] = plsc.load_gather(wtab, [n_vec])

        def task_body(t, carry):
            task = wid * tpw + t
            b = task // (C // CCH)
            c0 = (task - b * (C // CCH)) * CCH

            @pl.when(t > 0)
            def _():
                for k in range(K):
                    pltpu.async_copy(x_hbm.at[b, k, pl.ds(c0, CCH)],
                                     xin.at[k], si)

            for k in range(K):
                pltpu.make_async_copy(x_hbm.at[b, k, pl.ds(c0, CCH)],
                                      xin.at[k], si).wait()

            def slab_body(s, carry2):
                i = s // CCH
                c = s - i * CCH
                gs = t * (NCH * CCH) + s
                slot = lax.rem(gs, NBUF)
                ch = i * C + c0 + c

                @pl.when(gs >= NBUF)
                def _():
                    pltpu.make_async_copy(ob.at[slot], out_hbm.at[b, ch],
                                          so).wait()

                for n in range(NSEL):
                    sel_ix = jnp.full((LANES,), i * LANES + n, jnp.int32)
                    kv = plsc.load_gather(coltab, [sel_ix])
                    wv = plsc.load_gather(wseltab, [sel_ix])
                    k = kv[0]

                    @plsc.parallel_loop(0, RCHUNKS, unroll=UNROLL)
                    def rchunk(j, n=n, k=k, c=c, wv=wv, slot=slot):
                        v = xin[k, c, pl.ds(j * LANES, LANES)]
                        ob[slot, n, pl.ds(j * LANES, LANES)] = v * wv

                pltpu.async_copy(ob.at[slot], out_hbm.at[b, ch], so)
                return carry2

            lax.fori_loop(0, NCH * CCH, slab_body, 0)
            return carry

        lax.fori_loop(0, tpw, task_body, 0)

        for _ in range(NBUF):
            pltpu.make_async_copy(ob.at[0], out_hbm.at[0, 0], so).wait()

    return sc_call


_sc_call = _make_sc_call()


def kernel(X, partition_weights):
    wpad = jnp.concatenate(
        [partition_weights,
         jnp.full((NCH, LANES - K), jnp.inf, jnp.float32)], axis=1)
    xt = X.transpose(0, 3, 1, 2)
    out = _sc_call(xt, wpad.reshape(NCH * LANES))
    return out.transpose(0, 1, 3, 2)
